# baseline (speedup 1.0000x reference)
import functools

import jax
import jax.numpy as jnp
from jax import lax
from jax.experimental import pallas as pl
from jax.experimental.pallas import tpu as pltpu
from jax.experimental.pallas import tpu_sc as plsc

BATCH = 16384
EMBED = 64

_info = plsc.get_sparse_core_info()
_NC, _NS = _info.num_cores, _info.num_subcores
_NW = _NC * _NS
_B_PER_W = BATCH // _NW
_G = _B_PER_W // 16


def _make_lookup():
  mesh = plsc.VectorSubcoreMesh(core_axis_name="c", subcore_axis_name="s")

  @functools.partial(
      pl.kernel,
      mesh=mesh,
      out_type=jax.ShapeDtypeStruct((BATCH, EMBED), jnp.float32),
      scratch_types=[
          pltpu.VMEM((_B_PER_W,), jnp.int32),
          pltpu.VMEM((_B_PER_W, EMBED), jnp.float32),
          pltpu.SemaphoreType.DMA((16,)),
      ],
      compiler_params=pltpu.CompilerParams(use_tc_tiling_on_sc=True),
  )
  def lookup_kernel(word_hbm, emb_hbm, out_hbm, idx_v, rows_v, sem):
    wid = lax.axis_index("s") * _NC + lax.axis_index("c")
    base = wid * _B_PER_W
    pltpu.sync_copy(word_hbm.at[pl.ds(base, _B_PER_W)], idx_v)

    def fire(g):
      wv = idx_v[pl.ds(g * 16, 16)]
      for k in range(16):
        w = wv[k]
        pltpu.async_copy(emb_hbm.at[pl.ds(w, 1)],
                         rows_v.at[pl.ds(g * 16 + k, 1)], sem.at[k])

    def drain(g):
      for k in range(16):
        pltpu.make_async_copy(emb_hbm.at[pl.ds(0, 1)],
                              rows_v.at[pl.ds(g * 16 + k, 1)],
                              sem.at[k]).wait()

    fire(0)

    def group_body(g, _):
      fire(g)
      drain(g - 1)
      return _

    lax.fori_loop(1, _G, group_body, None)
    drain(_G - 1)

    pltpu.sync_copy(rows_v, out_hbm.at[pl.ds(base, _B_PER_W)])

  return lookup_kernel


_lookup = _make_lookup()


def kernel(word, emb):
  return _lookup(word, emb)


# speedup vs baseline: 1.7016x; 1.7016x over previous
import functools

import jax
import jax.numpy as jnp
from jax import lax
from jax.experimental import pallas as pl
from jax.experimental.pallas import tpu as pltpu
from jax.experimental.pallas import tpu_sc as plsc

BATCH = 16384
EMBED = 64
VOCAB = 1000000

_info = plsc.get_sparse_core_info()
_NC, _NS = _info.num_cores, _info.num_subcores
_NW = _NC * _NS
_TCW = 244
_NR = _TCW * 128
_CHUNK = 512
_NCH = _NR // _CHUNK
_TAIL0 = _NW * _NR
_TAIL1 = _TAIL0 + 512
_CAP = 128
_NGROUPS = BATCH // 16


def _make_lookup():
  mesh = plsc.VectorSubcoreMesh(core_axis_name="c", subcore_axis_name="s")

  @functools.partial(
      pl.kernel,
      mesh=mesh,
      out_type=jax.ShapeDtypeStruct((BATCH + 1, 128), jnp.float32),
      scratch_types=[
          pltpu.VMEM((BATCH,), jnp.int32),
          pltpu.VMEM((BATCH,), jnp.int32),
          pltpu.VMEM((EMBED, _CHUNK), jnp.float32),
          pltpu.VMEM((EMBED, _CHUNK), jnp.float32),
          pltpu.VMEM((_CAP, 128), jnp.float32),
          pltpu.VMEM((_CAP,), jnp.int32),
          pltpu.SemaphoreType.DMA((4,)),
      ],
      compiler_params=pltpu.CompilerParams(needs_layout_passes=False),
  )
  def lookup_kernel(word_hbm, embt_hbm, out_hbm, word_v, plist_v, buf0, buf1,
                    rows_v, bidx_v, sem):
    wid = lax.axis_index("s") * _NC + lax.axis_index("c")
    lo = wid * _NR
    lane = lax.iota(jnp.int32, 16)
    trash = jnp.full((16,), BATCH, jnp.int32)

    is30 = wid == _NW - 2
    lo2 = jnp.where(is30, _TAIL0, VOCAB)
    len2 = jnp.where(is30, 512, 0)

    pltpu.sync_copy(word_hbm.at[pl.ds(0, BATCH)], word_v)

    for t in range(_CAP // 16):
      plsc.store_scatter(bidx_v, [lane + t * 16], trash)

    def scan_body(t, off):
      wv = word_v[pl.ds(t * 16, 16)]
      wrel = wv - lo
      in1 = (wrel >= 0) & (wrel < _NR)
      wrel2 = wv - lo2 + _NR
      in2 = (wrel2 >= _NR) & (wrel2 < _NR + len2)
      mask = in1 | in2
      wr = jnp.where(in2, wrel2, wrel)
      ones = jnp.where(mask, 1, 0)
      pos = off + plsc.cumsum(ones) - 1
      packed = lax.shift_left(wr, 14) + (lane + t * 16)
      plsc.store_scatter(plist_v, [pos], packed, mask=mask)
      return off + plsc.all_reduce_population_count(mask)[0]

    cnt = lax.fori_loop(0, _NGROUPS, scan_body, jnp.int32(0))
    ngrp = lax.div(cnt + 15, jnp.int32(16))

    def fire_chunk(c, buf):
      col0 = pl.multiple_of(lo + c * _CHUNK, 128)
      pltpu.async_copy(embt_hbm.at[:, pl.ds(col0, _CHUNK)], buf,
                       sem.at[c % 2])

    def wait_chunk(c, buf):
      pltpu.make_async_copy(embt_hbm.at[:, pl.ds(0, _CHUNK)], buf,
                            sem.at[c % 2]).wait()

    def process_range(buf, wlo, wlen, slot):
      def grp_body(t, slot):
        pv = plist_v[pl.ds(t * 16, 16)]
        valid = (lane + t * 16) < cnt
        wrel = lax.shift_right_logical(pv, 14)
        inr = valid & (wrel >= wlo) & (wrel < wlo + wlen)
        pos = plsc.cumsum(jnp.where(inr, 1, 0)) - 1
        n = plsc.all_reduce_population_count(inr)[0]
        cw = jnp.where(inr, wrel - wlo, 0)
        cb = jnp.where(inr, pv & 16383, 0)

        def sel_body(j, slot):
          @pl.when(slot == _CAP)
          def _():
            pltpu.async_copy(rows_v, out_hbm.at[bidx_v], sem.at[2])
            pltpu.make_async_copy(rows_v, out_hbm.at[bidx_v],
                                  sem.at[2]).wait()
            for u in range(_CAP // 16):
              plsc.store_scatter(bidx_v, [lane + u * 16], trash)

          slot = jnp.where(slot == _CAP, 0, slot)
          jspl = jnp.full((16,), j, jnp.int32)
          hit = inr & (pos == jspl)
          wsel = jnp.max(jnp.where(hit, cw, 0))
          bsel = jnp.max(jnp.where(hit, cb, 0))
          wspl = jnp.full((16,), wsel, jnp.int32)
          for g in range(EMBED // 16):
            vals = plsc.load_gather(buf, [lane + g * 16, wspl])
            rows_v[slot, pl.ds(g * 16, 16)] = vals
          plsc.store_scatter(bidx_v, [jnp.full((16,), slot, jnp.int32)],
                             jnp.full((16,), bsel, jnp.int32),
                             mask=lane == 0)
          return slot + 1

        return lax.fori_loop(0, n, sel_body, slot)

      return lax.fori_loop(0, ngrp, grp_body, slot)

    fire_chunk(0, buf0)
    fire_chunk(1, buf1)
    slot = jnp.int32(0)

    def pair_body(k, slot):
      c0 = 2 * k
      wait_chunk(c0, buf0)
      slot = process_range(buf0, c0 * _CHUNK, _CHUNK, slot)

      @pl.when(c0 + 2 < _NCH)
      def _():
        fire_chunk(c0 + 2, buf0)

      wait_chunk(c0 + 1, buf1)
      slot = process_range(buf1, (c0 + 1) * _CHUNK, _CHUNK, slot)

      @pl.when(c0 + 3 < _NCH)
      def _():
        fire_chunk(c0 + 3, buf1)

      return slot

    slot = lax.fori_loop(0, _NCH // 2, pair_body, slot)
    wait_chunk(_NCH - 1, buf0)
    slot = process_range(buf0, (_NCH - 1) * _CHUNK, _CHUNK, slot)

    pltpu.async_copy(embt_hbm.at[:, pl.ds(_TAIL0, _CHUNK)], buf0, sem.at[0])
    pltpu.make_async_copy(embt_hbm.at[:, pl.ds(0, _CHUNK)], buf0,
                          sem.at[0]).wait()
    slot = process_range(buf0, _NR, len2, slot)

    pltpu.async_copy(rows_v, out_hbm.at[bidx_v], sem.at[2])
    pltpu.make_async_copy(rows_v, out_hbm.at[bidx_v], sem.at[2]).wait()

  return lookup_kernel


_lookup = _make_lookup()


def kernel(word, emb):
  out_padded = _lookup(word, jnp.transpose(emb))
  out_main = lax.slice(out_padded, (0, 0), (BATCH, EMBED))
  tail_tab = lax.slice(emb, (_TAIL1, 0), (VOCAB, EMBED))
  wt = jnp.clip(word - _TAIL1, 0, VOCAB - _TAIL1 - 1)
  tail_rows = jnp.take(tail_tab, wt, axis=0)
  return jnp.where((word >= _TAIL1)[:, None], tail_rows, out_main)
